# baseline (speedup 1.0000x reference)
import functools

import jax
import jax.numpy as jnp
from jax import lax
from jax.experimental import pallas as pl
from jax.experimental.pallas import tpu as pltpu
from jax.experimental.pallas import tpu_sc as plsc

B = 16384
N_FIELDS = 26
VOCAB = 100000
D = 64
GENRE = 10
NCOLS = N_FIELDS + GENRE
NOUT = N_FIELDS + 1
CB = 16
ROWS = CB * NOUT
L = 16


def _sc_call():
  info = plsc.get_sparse_core_info()
  nc, ns = info.num_cores, info.num_subcores
  nw = nc * ns
  per_w = B // nw
  n_chunks = per_w // CB
  mesh = plsc.VectorSubcoreMesh(core_axis_name="c", subcore_axis_name="s")

  @functools.partial(
      pl.kernel,
      mesh=mesh,
      compiler_params=pltpu.CompilerParams(
          use_tc_tiling_on_sc=True, needs_layout_passes=False),
      out_type=jax.ShapeDtypeStruct((B * NOUT, D), jnp.float32),
      scratch_types=[
          pltpu.VMEM((CB, NCOLS), jnp.int32),
          pltpu.VMEM((ROWS,), jnp.int32),
          pltpu.VMEM((ROWS,), jnp.int32),
          pltpu.VMEM((ROWS, 2 * D), jnp.float32),
          pltpu.VMEM((ROWS, D), jnp.float32),
          pltpu.VMEM((GENRE, D), jnp.float32),
          pltpu.VMEM((CB, D), jnp.float32),
          pltpu.SemaphoreType.DMA,
      ],
  )
  def k(x_hbm, tab_hbm, ge_hbm, out_hbm,
        x_v, idx_v, h_v, rows_v, out_v, ge_v, gsc_v, sem):
    wid = lax.axis_index("s") * nc + lax.axis_index("c")
    base = wid * per_w
    pltpu.sync_copy(ge_hbm, ge_v)
    lane = lax.iota(jnp.int32, L)

    def chunk(i, carry):
      row0 = base + i * CB
      pltpu.sync_copy(x_hbm.at[pl.ds(row0, CB)], x_v)
      for t in range(ROWS // L):
        pos0 = t * L
        j0 = pos0 // NOUT
        cut = NOUT * (j0 + 1) - pos0
        if cut <= L - 1:
          j = jnp.where(lane >= cut, j0 + 1, j0)
        else:
          j = jnp.full((L,), j0, jnp.int32)
        f = pos0 + lane - j * NOUT
        xv = plsc.load_gather(x_v, [j, jnp.minimum(f, NCOLS - 1)])
        idx = jnp.where(f < N_FIELDS, f * VOCAB + xv, 0)
        idx_v[pl.ds(t * L, L)] = lax.shift_right_logical(idx, 1)
        h_v[pl.ds(t * L, L)] = lax.bitwise_and(idx, 1)
      copies = []
      off = 0
      while off < ROWS:
        n = min(128, ROWS - off)
        copies.append(pltpu.async_copy(
            tab_hbm.at[idx_v.at[pl.ds(off, n)]],
            rows_v.at[pl.ds(off, n)], sem))
        off += n
      fg = jnp.minimum(lane + N_FIELDS, NCOLS - 1)
      for j in range(CB):
        jv = jnp.full((L,), j, jnp.int32)
        g = plsc.load_gather(x_v, [jv, fg]).astype(jnp.float32)
        g = jnp.where(lane < GENRE, g, 0.0)
        s = g
        for st in (1, 2, 4, 8):
          s = s + s.at[lane ^ st].get(mode="promise_in_bounds")
        w = g / s
        acc = [None] * (D // L)
        for kk in range(GENRE):
          wk = w.at[jnp.full((L,), kk, jnp.int32)].get(
              mode="promise_in_bounds")
          for c in range(D // L):
            term = wk * ge_v[kk, pl.ds(c * L, L)]
            acc[c] = term if acc[c] is None else acc[c] + term
        for c in range(D // L):
          gsc_v[j, pl.ds(c * L, L)] = acc[c]
      for cp in copies:
        cp.wait()
      def compact(r, cc):
        rv = jnp.zeros((L,), jnp.int32) + r
        hs = plsc.load_gather(h_v, [rv])
        for c in range(D // L):
          val = plsc.load_gather(rows_v, [rv, hs * D + c * L + lane])
          out_v[r, pl.ds(c * L, L)] = val
        return cc

      lax.fori_loop(0, ROWS, compact, 0)
      for j in range(CB):
        for c in range(D // L):
          out_v[j * NOUT + N_FIELDS, pl.ds(c * L, L)] = gsc_v[j, pl.ds(c * L, L)]
      pltpu.sync_copy(out_v, out_hbm.at[pl.ds(row0 * NOUT, ROWS)])
      return carry

    lax.fori_loop(0, n_chunks, chunk, 0)

  return k


def kernel(x, tables, genre_embed):
  tab_pairs = tables.reshape(N_FIELDS * VOCAB // 2, 2 * D)
  out = _sc_call()(x, tab_pairs, genre_embed)
  return out.reshape(B, NOUT, D)


# speedup vs baseline: 1.2403x; 1.2403x over previous
import functools

import jax
import jax.numpy as jnp
from jax import lax
from jax.experimental import pallas as pl
from jax.experimental.pallas import tpu as pltpu
from jax.experimental.pallas import tpu_sc as plsc

B = 16384
N_FIELDS = 26
VOCAB = 100000
D = 64
GENRE = 10
NCOLS = N_FIELDS + GENRE
NOUT = N_FIELDS + 1
CB = 16
ROWS = CB * NOUT
L = 16
_GROUPS = tuple((o, min(128, ROWS - o)) for o in range(0, ROWS, 128))


def _sc_call():
  info = plsc.get_sparse_core_info()
  nc, ns = info.num_cores, info.num_subcores
  nw = nc * ns
  per_w = B // nw
  n_chunks = per_w // CB
  mesh = plsc.VectorSubcoreMesh(core_axis_name="c", subcore_axis_name="s")

  @functools.partial(
      pl.kernel,
      mesh=mesh,
      compiler_params=pltpu.CompilerParams(
          use_tc_tiling_on_sc=False, needs_layout_passes=False),
      out_type=jax.ShapeDtypeStruct((B * NOUT, D), jnp.float32),
      scratch_types=[
          pltpu.VMEM((CB, NCOLS), jnp.int32),
          pltpu.VMEM((CB, NCOLS), jnp.int32),
          pltpu.VMEM((ROWS,), jnp.int32),
          pltpu.VMEM((ROWS,), jnp.int32),
          pltpu.VMEM((ROWS, D), jnp.float32),
          pltpu.VMEM((ROWS, D), jnp.float32),
          pltpu.VMEM((CB, D), jnp.float32),
          pltpu.VMEM((CB, D), jnp.float32),
          pltpu.VMEM((GENRE, D), jnp.float32),
          pltpu.SemaphoreType.DMA,
          pltpu.SemaphoreType.DMA,
      ],
  )
  def k(x_hbm, tab_hbm, ge_hbm, out_hbm,
        x_v0, x_v1, idx_v0, idx_v1, rows_v0, rows_v1, gsc_v0, gsc_v1,
        ge_v, sem0, sem1):
    wid = lax.axis_index("s") * nc + lax.axis_index("c")
    base = wid * per_w
    pltpu.sync_copy(ge_hbm, ge_v)
    lane = lax.iota(jnp.int32, L)
    fg = jnp.minimum(lane + N_FIELDS, NCOLS - 1)
    bufs = ((x_v0, idx_v0, rows_v0, gsc_v0, sem0),
            (x_v1, idx_v1, rows_v1, gsc_v1, sem1))

    def load_and_fire(ci, x_v, idx_v, rows_v, sem):
      row0 = base + ci * CB
      pltpu.sync_copy(x_hbm.at[pl.ds(row0, CB)], x_v)
      for t in range(ROWS // L):
        pos0 = t * L
        j0 = pos0 // NOUT
        cut = NOUT * (j0 + 1) - pos0
        if cut <= L - 1:
          j = jnp.where(lane >= cut, j0 + 1, j0)
        else:
          j = jnp.full((L,), j0, jnp.int32)
        f = pos0 + lane - j * NOUT
        xv = plsc.load_gather(x_v, [j, jnp.minimum(f, NCOLS - 1)])
        idx = jnp.where(f < N_FIELDS, f * VOCAB + xv, 0)
        idx_v[pl.ds(t * L, L)] = idx
      for o, n in _GROUPS:
        pltpu.async_copy(tab_hbm.at[idx_v.at[pl.ds(o, n)]],
                         rows_v.at[pl.ds(o, n)], sem)

    def drain(idx_v, rows_v, sem):
      for o, n in _GROUPS:
        pltpu.make_async_copy(tab_hbm.at[idx_v.at[pl.ds(o, n)]],
                              rows_v.at[pl.ds(o, n)], sem).wait()

    def genre(x_v, gsc_v):
      for j in range(CB):
        jv = jnp.full((L,), j, jnp.int32)
        g = plsc.load_gather(x_v, [jv, fg]).astype(jnp.float32)
        g = jnp.where(lane < GENRE, g, 0.0)
        s = g
        for st in (1, 2, 4, 8):
          s = s + s.at[lane ^ st].get(mode="promise_in_bounds")
        w = g / s
        acc = [None] * (D // L)
        for kk in range(GENRE):
          wk = w.at[jnp.full((L,), kk, jnp.int32)].get(
              mode="promise_in_bounds")
          for c in range(D // L):
            term = wk * ge_v[kk, pl.ds(c * L, L)]
            acc[c] = term if acc[c] is None else acc[c] + term
        for c in range(D // L):
          gsc_v[j, pl.ds(c * L, L)] = acc[c]

    def merge_and_store(ci, rows_v, gsc_v):
      for j in range(CB):
        for c in range(D // L):
          rows_v[j * NOUT + N_FIELDS, pl.ds(c * L, L)] = (
              gsc_v[j, pl.ds(c * L, L)])
      row0 = base + ci * CB
      pltpu.sync_copy(rows_v, out_hbm.at[pl.ds(row0 * NOUT, ROWS)])

    load_and_fire(0, bufs[0][0], bufs[0][1], bufs[0][2], bufs[0][4])

    def pair(k2, carry):
      for p in (0, 1):
        ci = 2 * k2 + p
        x_v, idx_v, rows_v, gsc_v, sem = bufs[p]
        nx_v, nidx_v, nrows_v, ngsc_v, nsem = bufs[1 - p]
        load_and_fire(jnp.minimum(ci + 1, n_chunks - 1),
                      nx_v, nidx_v, nrows_v, nsem)
        genre(x_v, gsc_v)
        drain(idx_v, rows_v, sem)
        merge_and_store(ci, rows_v, gsc_v)
      return carry

    lax.fori_loop(0, n_chunks // 2, pair, 0)
    drain(bufs[0][1], bufs[0][2], bufs[0][4])

  return k


def kernel(x, tables, genre_embed):
  tab_flat = tables.reshape(N_FIELDS * VOCAB, D)
  out = _sc_call()(x, tab_flat, genre_embed)
  return out.reshape(B, NOUT, D)


# speedup vs baseline: 1.2477x; 1.0060x over previous
import functools

import jax
import jax.numpy as jnp
from jax import lax
from jax.experimental import pallas as pl
from jax.experimental.pallas import tpu as pltpu
from jax.experimental.pallas import tpu_sc as plsc

B = 16384
N_FIELDS = 26
VOCAB = 100000
D = 64
GENRE = 10
NCOLS = N_FIELDS + GENRE
NOUT = N_FIELDS + 1
CB = 16
ROWS = CB * NOUT
L = 16


def _sc_call():
  info = plsc.get_sparse_core_info()
  nc, ns = info.num_cores, info.num_subcores
  nw = nc * ns
  per_w = B // nw
  n_chunks = per_w // CB
  mesh = plsc.VectorSubcoreMesh(core_axis_name="c", subcore_axis_name="s")

  @functools.partial(
      pl.kernel,
      mesh=mesh,
      compiler_params=pltpu.CompilerParams(use_tc_tiling_on_sc=False, needs_layout_passes=False),
      out_type=jax.ShapeDtypeStruct((B * NOUT, D), jnp.float32),
      scratch_types=[
          pltpu.VMEM((CB, NCOLS), jnp.int32),
          pltpu.VMEM((ROWS,), jnp.int32),
          pltpu.VMEM((ROWS, D), jnp.float32),
          pltpu.VMEM((GENRE, D), jnp.float32),
          pltpu.VMEM((CB, D), jnp.float32),
          pltpu.SemaphoreType.DMA,
      ],
  )
  def k(x_hbm, tab_hbm, ge_hbm, out_hbm,
        x_v, idx_v, rows_v, ge_v, gsc_v, sem):
    wid = lax.axis_index("s") * nc + lax.axis_index("c")
    base = wid * per_w
    pltpu.sync_copy(ge_hbm, ge_v)
    lane = lax.iota(jnp.int32, L)

    def chunk(i, carry):
      row0 = base + i * CB
      pltpu.sync_copy(x_hbm.at[pl.ds(row0, CB)], x_v)
      for t in range(ROWS // L):
        pos0 = t * L
        j0 = pos0 // NOUT
        cut = NOUT * (j0 + 1) - pos0
        if cut <= L - 1:
          j = jnp.where(lane >= cut, j0 + 1, j0)
        else:
          j = jnp.full((L,), j0, jnp.int32)
        f = pos0 + lane - j * NOUT
        xv = plsc.load_gather(x_v, [j, jnp.minimum(f, NCOLS - 1)])
        idx = jnp.where(f < N_FIELDS, f * VOCAB + xv, 0)
        idx_v[pl.ds(t * L, L)] = idx
      copies = []
      off = 0
      while off < ROWS:
        n = min(128, ROWS - off)
        copies.append(pltpu.async_copy(
            tab_hbm.at[idx_v.at[pl.ds(off, n)]],
            rows_v.at[pl.ds(off, n)], sem))
        off += n
      fg = jnp.minimum(lane + N_FIELDS, NCOLS - 1)
      for j in range(CB):
        jv = jnp.full((L,), j, jnp.int32)
        g = plsc.load_gather(x_v, [jv, fg]).astype(jnp.float32)
        g = jnp.where(lane < GENRE, g, 0.0)
        s = g
        for st in (1, 2, 4, 8):
          s = s + s.at[lane ^ st].get(mode="promise_in_bounds")
        w = g / s
        acc = [None] * (D // L)
        for kk in range(GENRE):
          wk = w.at[jnp.full((L,), kk, jnp.int32)].get(
              mode="promise_in_bounds")
          for c in range(D // L):
            term = wk * ge_v[kk, pl.ds(c * L, L)]
            acc[c] = term if acc[c] is None else acc[c] + term
        for c in range(D // L):
          gsc_v[j, pl.ds(c * L, L)] = acc[c]
      for cp in copies:
        cp.wait()
      for j in range(CB):
        for c in range(D // L):
          rows_v[j * NOUT + N_FIELDS, pl.ds(c * L, L)] = gsc_v[j, pl.ds(c * L, L)]
      pltpu.sync_copy(rows_v, out_hbm.at[pl.ds(row0 * NOUT, ROWS)])
      return carry

    lax.fori_loop(0, n_chunks, chunk, 0)

  return k


def kernel(x, tables, genre_embed):
  tab_flat = tables.reshape(N_FIELDS * VOCAB, D)
  out = _sc_call()(x, tab_flat, genre_embed)
  return out.reshape(B, NOUT, D)
